# baseline (speedup 1.0000x reference)
import jax
import jax.numpy as jnp
from jax import lax
from jax.experimental import pallas as pl
from jax.experimental.pallas import tpu as pltpu
from jax.experimental.pallas import tpu_sc as plsc

N = 10000
NP = 10240
E = 320000
D = 128
NC = 2
NS = 16
NW = NC * NS
EPT = E // NW
G = 80
NG = EPT // G
RPT = NP // NS
MAIN = (NG - 5) // 4


def _sc_body(x_hbm, src_hbm, dst_hbm, e_hbm, out_hbm,
             acc, sidx, didx, xr0, xr1, er0, er1, gsem, esem, ssem, isem):
    c = lax.axis_index("c")
    s = lax.axis_index("s")
    wid = s * NC + c
    XR = (xr0, xr1)
    ER = (er0, er1)
    zero = jnp.zeros((16,), jnp.float32)

    def issue_idx(g, q):
        base = wid * EPT + g * G
        pltpu.async_copy(src_hbm.at[pl.ds(base, G)], sidx.at[q], isem.at[q])
        pltpu.async_copy(dst_hbm.at[pl.ds(base, G)], didx.at[q], isem.at[q])

    def wait_idx(q):
        pltpu.make_async_copy(src_hbm.at[pl.ds(0, G)], sidx.at[q], isem.at[q]).wait()
        pltpu.make_async_copy(dst_hbm.at[pl.ds(0, G)], didx.at[q], isem.at[q]).wait()

    def issue_gather(q, b):
        pltpu.async_copy(x_hbm.at[sidx.at[q]], XR[b], gsem.at[b])

    def wait_gather(q, b):
        pltpu.make_async_copy(x_hbm.at[sidx.at[q]], XR[b], gsem.at[b]).wait()

    def issue_e(g, b):
        pltpu.async_copy(e_hbm.at[wid, g], ER[b], esem.at[b])

    def wait_e(b):
        pltpu.make_async_copy(e_hbm.at[wid, 0], ER[b], esem.at[b]).wait()

    def issue_scatter(q, b):
        pltpu.async_copy(ER[b], acc.at[didx.at[q]], ssem.at[b], add=True)

    def wait_scatter(q, b):
        pltpu.make_async_copy(ER[b], acc.at[didx.at[q]], ssem.at[b]).wait()

    def compute(b):
        xr, er = XR[b], ER[b]

        hi_mask = jnp.int32(-65536)

        def _row(r, carry):
            for j in range(D // 32):
                xv = xr[r, pl.ds(j * 16, 16)]
                xa = lax.bitcast_convert_type(xv << 16, jnp.float32)
                xb = lax.bitcast_convert_type(xv & hi_mask, jnp.float32)
                lo = pl.ds(j * 32, 16)
                hi = pl.ds(j * 32 + 16, 16)
                er[r, lo] = jnp.maximum(xa + er[r, lo], 0.0)
                er[r, hi] = jnp.maximum(xb + er[r, hi], 0.0)
            return carry
        lax.fori_loop(0, G, _row, None)

    def step(g, q, wait_next_idx=True, issue_next=True, issue_idx2=True):
        b = q % 2
        nb = 1 - b
        q1 = (q + 1) % 4
        q2 = (q + 2) % 4
        wait_gather(q, b)
        wait_e(b)
        if wait_next_idx:
            wait_idx(q1)
        wait_scatter((q + 3) % 4, nb)
        if issue_next:
            issue_gather(q1, nb)
            issue_e(g + 1, nb)
        if issue_idx2:
            issue_idx(g + 2, q2)
        compute(b)
        issue_scatter(q, b)

    def _zrow(r, carry):
        for j in range(D // 16):
            er0[r, pl.ds(j * 16, 16)] = zero
            er1[r, pl.ds(j * 16, 16)] = zero
        return carry
    lax.fori_loop(0, G, _zrow, None)
    for k in range(RPT // G):
        pltpu.sync_copy(er0, acc.at[pl.ds(s * RPT + k * G, G)])
    plsc.subcore_barrier()

    pltpu.sync_copy(dst_hbm.at[pl.ds(wid * EPT, G)], didx.at[3])
    pltpu.async_copy(er1, acc.at[didx.at[3]], ssem.at[1], add=True)
    issue_idx(0, 0)
    issue_idx(1, 1)
    wait_idx(0)
    issue_gather(0, 0)
    issue_e(0, 0)

    def _main(t, carry):
        g = t * 4
        for k in range(4):
            step(g + k, k)
        return carry
    lax.fori_loop(0, MAIN, _main, None)

    g0 = MAIN * 4
    for g in (g0, g0 + 1, g0 + 2):
        step(g, g % 4, issue_idx2=(g + 2 <= NG - 1))
    step(g0 + 3, (g0 + 3) % 4, issue_idx2=False)
    step(g0 + 4, (g0 + 4) % 4,
         wait_next_idx=False, issue_next=False, issue_idx2=False)
    wait_scatter((g0 + 4) % 4, (g0 + 4) % 2)

    plsc.subcore_barrier()
    pltpu.sync_copy(acc.at[pl.ds(s * RPT, RPT)],
                    out_hbm.at[c, pl.ds(s * RPT, RPT)])


def _sc_partials(x, src1, dst1, e4):
    mesh = plsc.VectorSubcoreMesh(core_axis_name="c", subcore_axis_name="s")
    return pl.kernel(
        _sc_body,
        out_type=jax.ShapeDtypeStruct((NC, NP, D), jnp.float32),
        mesh=mesh,
        scratch_types=[
            pltpu.VMEM_SHARED((NP, D), jnp.float32),
            pltpu.VMEM((4, G), jnp.int32),
            pltpu.VMEM((4, G), jnp.int32),
            pltpu.VMEM((G, D // 2), jnp.int32),
            pltpu.VMEM((G, D // 2), jnp.int32),
            pltpu.VMEM((G, D), jnp.float32),
            pltpu.VMEM((G, D), jnp.float32),
            pltpu.SemaphoreType.DMA((2,)),
            pltpu.SemaphoreType.DMA((2,)),
            pltpu.SemaphoreType.DMA((2,)),
            pltpu.SemaphoreType.DMA((4,)),
        ],
        compiler_params=pltpu.CompilerParams(use_tc_tiling_on_sc=False),
    )(x, src1, dst1, e4)


def _combine_body(x_ref, p_ref, o_ref):
    o_ref[...] = x_ref[...] + p_ref[0] + p_ref[1]


def _combine(x, partials):
    blk = 1000
    return pl.pallas_call(
        _combine_body,
        out_shape=jax.ShapeDtypeStruct((N, D), jnp.float32),
        grid=(N // blk,),
        in_specs=[
            pl.BlockSpec((blk, D), lambda i: (i, 0)),
            pl.BlockSpec((NC, blk, D), lambda i: (0, i, 0)),
        ],
        out_specs=pl.BlockSpec((blk, D), lambda i: (i, 0)),
    )(x, partials)


@jax.jit
def kernel(node_inputs, edge_index, edge_inputs):
    src1 = edge_index[0]
    dst1 = edge_index[1]
    e4 = edge_inputs.reshape(NW, NG, G, D)
    xb = node_inputs.astype(jnp.bfloat16)
    xp = xb.reshape(N, D // 32, 2, 16).swapaxes(2, 3).reshape(N, D // 2, 2)
    xi = jax.lax.bitcast_convert_type(xp, jnp.int32)
    partials = _sc_partials(xi, src1, dst1, e4)
    return _combine(node_inputs, partials)


# speedup vs baseline: 1.5493x; 1.5493x over previous
import jax
import jax.numpy as jnp
from jax import lax
from jax.experimental import pallas as pl
from jax.experimental.pallas import tpu as pltpu
from jax.experimental.pallas import tpu_sc as plsc

N = 10000
NP = 10240
E = 320000
D = 128
NC = 2
NS = 16
NW = NC * NS
EPT = E // NW
G = 80
NG = EPT // G
RPT = NP // NS
MAIN = (NG - 5) // 4


def _sc_body(x_hbm, idx_hbm, e_hbm, out_hbm,
             acc, idxr, xr0, xr1, er0, er1, gsem, esem, ssem, isem):
    c = lax.axis_index("c")
    s = lax.axis_index("s")
    wid = s * NC + c
    XR = (xr0, xr1)
    ER = (er0, er1)
    zero = jnp.zeros((16,), jnp.float32)

    def issue_idx(g, q):
        pltpu.async_copy(idx_hbm.at[wid, g], idxr.at[q], isem.at[q])

    def wait_idx(q):
        pltpu.make_async_copy(idx_hbm.at[wid, 0], idxr.at[q], isem.at[q]).wait()

    def issue_gather(q, b):
        pltpu.async_copy(x_hbm.at[idxr.at[q, 0]], XR[b], gsem.at[b])

    def wait_gather(q, b):
        pltpu.make_async_copy(x_hbm.at[idxr.at[q, 0]], XR[b], gsem.at[b]).wait()

    def issue_e(g, b):
        pltpu.async_copy(e_hbm.at[wid, g], ER[b], esem.at[b])

    def wait_e(b):
        pltpu.make_async_copy(e_hbm.at[wid, 0], ER[b], esem.at[b]).wait()

    def issue_scatter(q, b):
        pltpu.async_copy(XR[b], acc.at[idxr.at[q, 1]], ssem.at[b], add=True)

    def wait_scatter(q, b):
        pltpu.make_async_copy(XR[b], acc.at[idxr.at[q, 1]], ssem.at[b]).wait()

    def compute(b):
        xr, er = XR[b], ER[b]

        def _row(r, carry):
            for j in range(D // 16):
                sl = pl.ds(j * 16, 16)
                xr[r, sl] = jnp.maximum(xr[r, sl] + er[r, sl], 0.0)
            return carry
        lax.fori_loop(0, G, _row, None)

    def step(g, q, wait_next_idx=True, issue_next=True, issue_idx2=True):
        b = q % 2
        nb = 1 - b
        q1 = (q + 1) % 4
        q2 = (q + 2) % 4
        wait_gather(q, b)
        wait_e(b)
        if wait_next_idx:
            wait_idx(q1)
        wait_scatter((q + 3) % 4, nb)
        if issue_next:
            issue_gather(q1, nb)
            issue_e(g + 1, nb)
        if issue_idx2:
            issue_idx(g + 2, q2)
        compute(b)
        issue_scatter(q, b)

    issue_idx(0, 0)
    issue_idx(1, 1)

    def _zrow(r, carry):
        for j in range(D // 16):
            xr0[r, pl.ds(j * 16, 16)] = zero
            xr1[r, pl.ds(j * 16, 16)] = zero
        return carry
    lax.fori_loop(0, G, _zrow, None)
    for k in range(RPT // G):
        pltpu.sync_copy(xr0, acc.at[pl.ds(s * RPT + k * G, G)])
    plsc.subcore_barrier()

    wait_idx(0)
    pltpu.async_copy(xr1, acc.at[idxr.at[0, 1]], ssem.at[1], add=True)
    issue_gather(0, 0)
    issue_e(0, 0)

    def _main(t, carry):
        g = t * 4
        for k in range(4):
            step(g + k, k)
        return carry
    lax.fori_loop(0, MAIN, _main, None)

    g0 = MAIN * 4
    for g in (g0, g0 + 1, g0 + 2):
        step(g, g % 4, issue_idx2=(g + 2 <= NG - 1))
    step(g0 + 3, (g0 + 3) % 4, issue_idx2=False)
    step(g0 + 4, (g0 + 4) % 4,
         wait_next_idx=False, issue_next=False, issue_idx2=False)
    wait_scatter((g0 + 4) % 4, (g0 + 4) % 2)

    plsc.subcore_barrier()
    pltpu.sync_copy(acc.at[pl.ds(s * RPT, RPT)],
                    out_hbm.at[c, pl.ds(s * RPT, RPT)])


def _sc_partials(x, idx4, e4):
    mesh = plsc.VectorSubcoreMesh(core_axis_name="c", subcore_axis_name="s")
    return pl.kernel(
        _sc_body,
        out_type=jax.ShapeDtypeStruct((NC, NP, D), jnp.float32),
        mesh=mesh,
        scratch_types=[
            pltpu.VMEM_SHARED((NP, D), jnp.float32),
            pltpu.VMEM((4, 2, G), jnp.int32),
            pltpu.VMEM((G, D), jnp.float32),
            pltpu.VMEM((G, D), jnp.float32),
            pltpu.VMEM((G, D), jnp.float32),
            pltpu.VMEM((G, D), jnp.float32),
            pltpu.SemaphoreType.DMA((2,)),
            pltpu.SemaphoreType.DMA((2,)),
            pltpu.SemaphoreType.DMA((2,)),
            pltpu.SemaphoreType.DMA((4,)),
        ],
    )(x, idx4, e4)


def _combine_body(x_ref, p_ref, o_ref):
    o_ref[...] = x_ref[...] + p_ref[0] + p_ref[1]


def _combine(x, partials):
    blk = 1000
    return pl.pallas_call(
        _combine_body,
        out_shape=jax.ShapeDtypeStruct((N, D), jnp.float32),
        grid=(N // blk,),
        in_specs=[
            pl.BlockSpec((blk, D), lambda i: (i, 0)),
            pl.BlockSpec((NC, blk, D), lambda i: (0, i, 0)),
        ],
        out_specs=pl.BlockSpec((blk, D), lambda i: (i, 0)),
    )(x, partials)


@jax.jit
def kernel(node_inputs, edge_index, edge_inputs):
    idx4 = edge_index.reshape(2, NW, NG, G).transpose(1, 2, 0, 3)
    e4 = edge_inputs.reshape(NW, NG, G, D)
    partials = _sc_partials(node_inputs, idx4, e4)
    return _combine(node_inputs, partials)


# speedup vs baseline: 1.6202x; 1.0458x over previous
import jax
import jax.numpy as jnp
from jax import lax
from jax.experimental import pallas as pl
from jax.experimental.pallas import tpu as pltpu
from jax.experimental.pallas import tpu_sc as plsc

N = 10000
NP = 10240
E = 320000
D = 128
NC = 2
NS = 16
NW = NC * NS
EPT = E // NW
G = 40
NG = EPT // G
NB = 4
NQ = 8
RPT = NP // NS
MAIN = (NG - 10) // NQ


def _sc_body(x_hbm, idx_hbm, e_hbm, out_hbm, acc, idxr,
             xr0, xr1, xr2, xr3, er0, er1, er2, er3,
             gsem, esem, ssem, isem):
    c = lax.axis_index("c")
    s = lax.axis_index("s")
    wid = s * NC + c
    XR = (xr0, xr1, xr2, xr3)
    ER = (er0, er1, er2, er3)
    zero = jnp.zeros((16,), jnp.float32)

    def issue_idx(g, q):
        pltpu.async_copy(idx_hbm.at[wid, g], idxr.at[q], isem.at[q])

    def wait_idx(q):
        pltpu.make_async_copy(idx_hbm.at[wid, 0], idxr.at[q], isem.at[q]).wait()

    def issue_gather(q, b):
        pltpu.async_copy(x_hbm.at[idxr.at[q, 0]], XR[b], gsem.at[b])

    def wait_gather(q, b):
        pltpu.make_async_copy(x_hbm.at[idxr.at[q, 0]], XR[b], gsem.at[b]).wait()

    def issue_e(g, b):
        pltpu.async_copy(e_hbm.at[wid, g], ER[b], esem.at[b])

    def wait_e(b):
        pltpu.make_async_copy(e_hbm.at[wid, 0], ER[b], esem.at[b]).wait()

    def issue_scatter(q, b):
        pltpu.async_copy(XR[b], acc.at[idxr.at[q, 1]], ssem.at[b], add=True)

    def wait_scatter(q, b):
        pltpu.make_async_copy(XR[b], acc.at[idxr.at[q, 1]], ssem.at[b]).wait()

    def compute(b):
        xr, er = XR[b], ER[b]

        def _row(r, carry):
            for j in range(D // 16):
                sl = pl.ds(j * 16, 16)
                xr[r, sl] = jnp.maximum(xr[r, sl] + er[r, sl], 0.0)
            return carry
        lax.fori_loop(0, G, _row, None)

    def step(g, k, wait_next_idx=True, issue_next=True, issue_idx4=True):
        b = k % NB
        q = k % NQ
        b2 = (k + 2) % NB
        q2 = (k + 2) % NQ
        q4 = (k + 4) % NQ
        wait_gather(q, b)
        wait_e(b)
        if wait_next_idx:
            wait_idx(q2)
        wait_scatter((k + 6) % NQ, b2)
        if issue_next:
            issue_gather(q2, b2)
            issue_e(g + 2, b2)
        if issue_idx4:
            issue_idx(g + 4, q4)
        compute(b)
        issue_scatter(q, b)

    for q in range(4):
        issue_idx(q, q)

    def _zrow(r, carry):
        for j in range(D // 16):
            er2[r, pl.ds(j * 16, 16)] = zero
            er3[r, pl.ds(j * 16, 16)] = zero
        return carry
    lax.fori_loop(0, G, _zrow, None)
    for k in range(RPT // G):
        pltpu.sync_copy(ER[2 + (k % 2)], acc.at[pl.ds(s * RPT + k * G, G)])
    plsc.subcore_barrier()

    wait_idx(0)
    pltpu.async_copy(er2, acc.at[idxr.at[0, 1]], ssem.at[2], add=True)
    pltpu.async_copy(er3, acc.at[idxr.at[0, 1]], ssem.at[3], add=True)
    issue_gather(0, 0)
    issue_e(0, 0)
    wait_idx(1)
    issue_gather(1, 1)
    issue_e(1, 1)

    def _main(t, carry):
        g = t * NQ
        for k in range(NQ):
            step(g + k, k)
        return carry
    lax.fori_loop(0, MAIN, _main, None)

    g0 = MAIN * NQ
    for g in range(g0, NG):
        step(g, g % NQ,
             wait_next_idx=(g + 2 <= NG - 1),
             issue_next=(g + 2 <= NG - 1),
             issue_idx4=(g + 4 <= NG - 1))
    wait_scatter((NG - 2) % NQ, (NG - 2) % NB)
    wait_scatter((NG - 1) % NQ, (NG - 1) % NB)

    plsc.subcore_barrier()
    pltpu.sync_copy(acc.at[pl.ds(s * RPT, RPT)],
                    out_hbm.at[c, pl.ds(s * RPT, RPT)])


def _sc_partials(x, idx4, e4):
    mesh = plsc.VectorSubcoreMesh(core_axis_name="c", subcore_axis_name="s")
    return pl.kernel(
        _sc_body,
        out_type=jax.ShapeDtypeStruct((NC, NP, D), jnp.float32),
        mesh=mesh,
        scratch_types=[
            pltpu.VMEM_SHARED((NP, D), jnp.float32),
            pltpu.VMEM((NQ, 2, G), jnp.int32),
            pltpu.VMEM((G, D), jnp.float32),
            pltpu.VMEM((G, D), jnp.float32),
            pltpu.VMEM((G, D), jnp.float32),
            pltpu.VMEM((G, D), jnp.float32),
            pltpu.VMEM((G, D), jnp.float32),
            pltpu.VMEM((G, D), jnp.float32),
            pltpu.VMEM((G, D), jnp.float32),
            pltpu.VMEM((G, D), jnp.float32),
            pltpu.SemaphoreType.DMA((NB,)),
            pltpu.SemaphoreType.DMA((NB,)),
            pltpu.SemaphoreType.DMA((NB,)),
            pltpu.SemaphoreType.DMA((NQ,)),
        ],
    )(x, idx4, e4)


def _combine_body(x_ref, p_ref, o_ref):
    o_ref[...] = x_ref[...] + p_ref[0] + p_ref[1]


def _combine(x, partials):
    blk = 1000
    return pl.pallas_call(
        _combine_body,
        out_shape=jax.ShapeDtypeStruct((N, D), jnp.float32),
        grid=(N // blk,),
        in_specs=[
            pl.BlockSpec((blk, D), lambda i: (i, 0)),
            pl.BlockSpec((NC, blk, D), lambda i: (0, i, 0)),
        ],
        out_specs=pl.BlockSpec((blk, D), lambda i: (i, 0)),
    )(x, partials)


@jax.jit
def kernel(node_inputs, edge_index, edge_inputs):
    idx4 = edge_index.reshape(2, NW, NG, G).transpose(1, 2, 0, 3)
    e4 = edge_inputs.reshape(NW, NG, G, D)
    partials = _sc_partials(node_inputs, idx4, e4)
    return _combine(node_inputs, partials)


# speedup vs baseline: 1.6224x; 1.0013x over previous
import jax
import jax.numpy as jnp
from jax import lax
from jax.experimental import pallas as pl
from jax.experimental.pallas import tpu as pltpu
from jax.experimental.pallas import tpu_sc as plsc

N = 10000
NP = 10240
E = 320000
D = 128
NC = 2
NS = 16
NW = NC * NS
EPT = E // NW
G = 40
NG = EPT // G
NB = 4
NQ = 8
RPT = NP // NS
MAIN = (NG - 10) // NQ


def _sc_body(x_hbm, idx_hbm, e_hbm, out_hbm, acc, idxr,
             xr0, xr1, xr2, xr3, er0, er1, er2, er3,
             gsem, esem, ssem, isem):
    c = lax.axis_index("c")
    s = lax.axis_index("s")
    wid = s * NC + c
    XR = (xr0, xr1, xr2, xr3)
    ER = (er0, er1, er2, er3)
    zero = jnp.zeros((16,), jnp.float32)

    def issue_idx(g, q):
        pltpu.async_copy(idx_hbm.at[wid, g], idxr.at[q], isem.at[q])

    def wait_idx(q):
        pltpu.make_async_copy(idx_hbm.at[wid, 0], idxr.at[q], isem.at[q]).wait()

    def issue_gather(q, b):
        pltpu.async_copy(x_hbm.at[idxr.at[q, 0]], XR[b], gsem.at[b])

    def wait_gather(q, b):
        pltpu.make_async_copy(x_hbm.at[idxr.at[q, 0]], XR[b], gsem.at[b]).wait()

    def issue_e(g, b):
        pltpu.async_copy(e_hbm.at[wid, g], ER[b], esem.at[b])

    def wait_e(b):
        pltpu.make_async_copy(e_hbm.at[wid, 0], ER[b], esem.at[b]).wait()

    def issue_scatter(q, b):
        pltpu.async_copy(XR[b], acc.at[idxr.at[q, 1]], ssem.at[b], add=True)

    def wait_scatter(q, b):
        pltpu.make_async_copy(XR[b], acc.at[idxr.at[q, 1]], ssem.at[b]).wait()

    def compute(b):
        xr, er = XR[b], ER[b]

        def _row(r, carry):
            for j in range(D // 16):
                sl = pl.ds(j * 16, 16)
                xr[r, sl] = jnp.maximum(xr[r, sl] + er[r, sl], 0.0)
            return carry
        lax.fori_loop(0, G, _row, None)

    def step(g, k, wait_next_idx=True, issue_next=True, issue_idx4=True):
        b = k % NB
        q = k % NQ
        b2 = (k + 2) % NB
        q2 = (k + 2) % NQ
        q4 = (k + 4) % NQ
        wait_gather(q, b)
        wait_e(b)
        if wait_next_idx:
            wait_idx(q2)
        wait_scatter((k + 6) % NQ, b2)
        if issue_next:
            issue_gather(q2, b2)
            issue_e(g + 2, b2)
        if issue_idx4:
            issue_idx(g + 4, q4)
        compute(b)
        issue_scatter(q, b)

    for q in range(4):
        issue_idx(q, q)
    wait_idx(0)
    issue_gather(0, 0)
    issue_e(0, 0)
    wait_idx(1)
    issue_gather(1, 1)
    issue_e(1, 1)

    def _zrow(r, carry):
        for j in range(D // 16):
            er2[r, pl.ds(j * 16, 16)] = zero
            er3[r, pl.ds(j * 16, 16)] = zero
        return carry
    lax.fori_loop(0, G, _zrow, None)
    for k in range(RPT // G):
        pltpu.sync_copy(ER[2 + (k % 2)], acc.at[pl.ds(s * RPT + k * G, G)])
    plsc.subcore_barrier()

    pltpu.async_copy(er2, acc.at[idxr.at[0, 1]], ssem.at[2], add=True)
    pltpu.async_copy(er3, acc.at[idxr.at[0, 1]], ssem.at[3], add=True)

    def _main(t, carry):
        g = t * NQ
        for k in range(NQ):
            step(g + k, k)
        return carry
    lax.fori_loop(0, MAIN, _main, None)

    g0 = MAIN * NQ
    for g in range(g0, NG):
        step(g, g % NQ,
             wait_next_idx=(g + 2 <= NG - 1),
             issue_next=(g + 2 <= NG - 1),
             issue_idx4=(g + 4 <= NG - 1))
    wait_scatter((NG - 2) % NQ, (NG - 2) % NB)
    wait_scatter((NG - 1) % NQ, (NG - 1) % NB)

    plsc.subcore_barrier()
    pltpu.sync_copy(acc.at[pl.ds(s * RPT, RPT)],
                    out_hbm.at[c, pl.ds(s * RPT, RPT)])


def _sc_partials(x, idx4, e4):
    mesh = plsc.VectorSubcoreMesh(core_axis_name="c", subcore_axis_name="s")
    return pl.kernel(
        _sc_body,
        out_type=jax.ShapeDtypeStruct((NC, NP, D), jnp.float32),
        mesh=mesh,
        scratch_types=[
            pltpu.VMEM_SHARED((NP, D), jnp.float32),
            pltpu.VMEM((NQ, 2, G), jnp.int32),
            pltpu.VMEM((G, D), jnp.float32),
            pltpu.VMEM((G, D), jnp.float32),
            pltpu.VMEM((G, D), jnp.float32),
            pltpu.VMEM((G, D), jnp.float32),
            pltpu.VMEM((G, D), jnp.float32),
            pltpu.VMEM((G, D), jnp.float32),
            pltpu.VMEM((G, D), jnp.float32),
            pltpu.VMEM((G, D), jnp.float32),
            pltpu.SemaphoreType.DMA((NB,)),
            pltpu.SemaphoreType.DMA((NB,)),
            pltpu.SemaphoreType.DMA((NB,)),
            pltpu.SemaphoreType.DMA((NQ,)),
        ],
    )(x, idx4, e4)


def _combine_body(x_ref, p_ref, o_ref):
    o_ref[...] = x_ref[...] + p_ref[0] + p_ref[1]


def _combine(x, partials):
    blk = 1000
    return pl.pallas_call(
        _combine_body,
        out_shape=jax.ShapeDtypeStruct((N, D), jnp.float32),
        grid=(N // blk,),
        in_specs=[
            pl.BlockSpec((blk, D), lambda i: (i, 0)),
            pl.BlockSpec((NC, blk, D), lambda i: (0, i, 0)),
        ],
        out_specs=pl.BlockSpec((blk, D), lambda i: (i, 0)),
    )(x, partials)


@jax.jit
def kernel(node_inputs, edge_index, edge_inputs):
    idx4 = edge_index.reshape(2, NW, NG, G).transpose(1, 2, 0, 3)
    e4 = edge_inputs.reshape(NW, NG, G, D)
    partials = _sc_partials(node_inputs, idx4, e4)
    return _combine(node_inputs, partials)
